# Initial kernel scaffold; baseline (speedup 1.0000x reference)
import jax
import jax.numpy as jnp
from jax.experimental import pallas as pl


def kernel(x0, x1, x2, x3, w1, b1):
    raise NotImplementedError("write your pallas kernel here")


# speedup vs baseline: 1.3549x; 1.3549x over previous
import jax
import jax.numpy as jnp
from jax.experimental import pallas as pl
from jax.experimental.pallas import tpu as pltpu

_B = 8
_CIN = 128
_S = 4
_LANE = 128


def _sum_kernel(x0_ref, x1_ref, x2_ref, x3_ref,
                a0_ref, a1_ref, a2_ref, a3_ref):
    h = pl.program_id(0)

    @pl.when(h == 0)
    def _():
        a0_ref[...] = jnp.zeros_like(a0_ref)
        a1_ref[...] = jnp.zeros_like(a1_ref)
        a2_ref[...] = jnp.zeros_like(a2_ref)
        a3_ref[...] = jnp.zeros_like(a3_ref)

    for x_ref, a_ref in ((x0_ref, a0_ref), (x1_ref, a1_ref),
                         (x2_ref, a2_ref), (x3_ref, a3_ref)):
        x = x_ref[...]
        thw = x.shape[-1]
        acc = x[:, 0:_LANE]
        for j in range(1, thw // _LANE):
            acc = acc + x[:, j * _LANE:(j + 1) * _LANE]
        a_ref[...] = a_ref[...] + acc


def _apply_kernel(w_ref, x0_ref, x1_ref, x2_ref, x3_ref,
                  o0_ref, o1_ref, o2_ref, o3_ref):
    w = w_ref[...]
    for x_ref, o_ref in ((x0_ref, o0_ref), (x1_ref, o1_ref),
                         (x2_ref, o2_ref), (x3_ref, o3_ref)):
        for k in range(_B):
            xk = x_ref[k * _CIN:(k + 1) * _CIN, :]
            o_ref[k * _B:(k + 1) * _B, :] = jnp.dot(
                w, xk, preferred_element_type=jnp.float32)


def kernel(x0, x1, x2, x3, w1, b1):
    B, cin, H, W = x0.shape
    HW = H * W
    R = B * cin
    xs2d = [x.reshape(R, HW) for x in (x0, x1, x2, x3)]

    thw_a = 512
    grid_a = HW // thw_a
    acc_shape = jax.ShapeDtypeStruct((R, _LANE), jnp.float32)
    sums = pl.pallas_call(
        _sum_kernel,
        out_shape=[acc_shape] * _S,
        grid=(grid_a,),
        in_specs=[pl.BlockSpec((R, thw_a), lambda h: (0, h))] * _S,
        out_specs=[pl.BlockSpec((R, _LANE), lambda h: (0, 0))] * _S,
        compiler_params=pltpu.CompilerParams(
            dimension_semantics=("arbitrary",),
            vmem_limit_bytes=48 * 1024 * 1024,
        ),
    )(*xs2d)

    xmean = (jnp.stack([a.sum(axis=1) for a in sums])
             .reshape(_S, B, cin).transpose(1, 0, 2).reshape(B, _S * cin)
             * (1.0 / HW))
    wts = xmean @ w1.T + b1[None, :]

    thw_b = 512
    grid_b = HW // thw_b
    out_shape = jax.ShapeDtypeStruct((B * B, HW), jnp.float32)
    outs = pl.pallas_call(
        _apply_kernel,
        out_shape=[out_shape] * _S,
        grid=(grid_b,),
        in_specs=[pl.BlockSpec((B, cin), lambda h: (0, 0))]
                 + [pl.BlockSpec((R, thw_b), lambda h: (0, h))] * _S,
        out_specs=[pl.BlockSpec((B * B, thw_b), lambda h: (0, h))] * _S,
        compiler_params=pltpu.CompilerParams(
            dimension_semantics=("arbitrary",),
            vmem_limit_bytes=48 * 1024 * 1024,
        ),
    )(wts, *xs2d)

    return [o.reshape(B, B, H, W) for o in outs]


# speedup vs baseline: 1.4658x; 1.0818x over previous
import functools

import jax
import jax.numpy as jnp
from jax.experimental import pallas as pl
from jax.experimental.pallas import tpu as pltpu

_B = 8
_CIN = 128
_S = 4
_LANE = 128
_THW = 512


def _fused_kernel(w1_ref, b1_ref, x0_ref, x1_ref, x2_ref, x3_ref,
                  o0_ref, o1_ref, o2_ref, o3_ref,
                  acc_ref, wts_ref, *, ht, thw, hw):
    i = pl.program_id(0)
    x_refs = (x0_ref, x1_ref, x2_ref, x3_ref)
    o_refs = (o0_ref, o1_ref, o2_ref, o3_ref)

    @pl.when(i == 0)
    def _():
        acc_ref[...] = jnp.zeros_like(acc_ref)

    @pl.when(i < ht)
    def _():
        for s in range(_S):
            x = x_refs[s][...].astype(jnp.float32)
            part = x[:, 0:_LANE]
            for j in range(1, thw // _LANE):
                part = part + x[:, j * _LANE:(j + 1) * _LANE]
            acc_ref[s] = acc_ref[s] + part

    @pl.when(i == ht)
    def _():
        b1 = b1_ref[...]
        wts = jnp.broadcast_to(b1, (_B, _CIN)).astype(jnp.float32)
        for s in range(_S):
            m_s = jnp.sum(acc_ref[s].reshape(_B, _CIN, _LANE), axis=2)
            m_s = m_s * (1.0 / hw)
            w1_s = w1_ref[:, s * _CIN:(s + 1) * _CIN]
            wts = wts + jax.lax.dot_general(
                m_s, w1_s, (((1,), (1,)), ((), ())),
                preferred_element_type=jnp.float32)
        wts_ref[...] = wts

    @pl.when(i >= ht)
    def _():
        w = wts_ref[...].astype(jnp.bfloat16)
        for s in range(_S):
            for k in range(_B):
                xk = x_refs[s][k * _CIN:(k + 1) * _CIN, :]
                o_refs[s][k * _B:(k + 1) * _B, :] = jnp.dot(
                    w, xk, preferred_element_type=jnp.float32)


def kernel(x0, x1, x2, x3, w1, b1):
    B, cin, H, W = x0.shape
    HW = H * W
    R = B * cin
    thw = min(_THW, HW)
    ht = HW // thw
    xs_c = [x.reshape(R, HW).astype(jnp.bfloat16) for x in (x0, x1, x2, x3)]

    body = functools.partial(_fused_kernel, ht=ht, thw=thw, hw=HW)

    def in_idx(i):
        return (0, jnp.where(i < ht, i, i - ht))

    def out_idx(i):
        return (0, jnp.where(i < ht, 0, i - ht))

    out_shape = jax.ShapeDtypeStruct((B * B, HW), jnp.float32)
    outs = pl.pallas_call(
        body,
        out_shape=[out_shape] * _S,
        grid=(2 * ht,),
        in_specs=[
            pl.BlockSpec((cin, _S * cin), lambda i: (0, 0)),
            pl.BlockSpec((1, cin), lambda i: (0, 0)),
        ] + [pl.BlockSpec((R, thw), in_idx)] * _S,
        out_specs=[pl.BlockSpec((B * B, thw), out_idx)] * _S,
        scratch_shapes=[
            pltpu.VMEM((_S, R, _LANE), jnp.float32),
            pltpu.VMEM((_B, _CIN), jnp.float32),
        ],
        compiler_params=pltpu.CompilerParams(
            dimension_semantics=("arbitrary",),
            vmem_limit_bytes=48 * 1024 * 1024,
        ),
    )(w1, b1.reshape(1, cin), *xs_c)

    return [o.reshape(B, B, H, W) for o in outs]


# speedup vs baseline: 1.5057x; 1.0272x over previous
import functools

import jax
import jax.numpy as jnp
from jax.experimental import pallas as pl
from jax.experimental.pallas import tpu as pltpu

_B = 8
_CIN = 128
_S = 4
_LANE = 128
_THW = 512


def _fused_kernel(w1_ref, b1_ref, x0_ref, x1_ref, x2_ref, x3_ref,
                  o0_ref, o1_ref, o2_ref, o3_ref,
                  acc_ref, wts_ref, *, ht, thw, hw, wdim):
    i = pl.program_id(0)
    x_refs = (x0_ref, x1_ref, x2_ref, x3_ref)
    o_refs = (o0_ref, o1_ref, o2_ref, o3_ref)

    @pl.when(i == 0)
    def _():
        acc_ref[...] = jnp.zeros_like(acc_ref)

    @pl.when(i < ht)
    def _():
        for s in range(_S):
            x = x_refs[s][...].astype(jnp.float32)
            part = x[:, 0:_LANE]
            for j in range(1, thw // _LANE):
                part = part + x[:, j * _LANE:(j + 1) * _LANE]
            acc_ref[s] = acc_ref[s] + part

    @pl.when(i == ht)
    def _():
        b1 = b1_ref[...]
        wts = jnp.broadcast_to(b1, (_B, _CIN)).astype(jnp.float32)
        for s in range(_S):
            m_s = jnp.sum(acc_ref[s].reshape(_B, _CIN, _LANE), axis=2)
            m_s = m_s * (1.0 / hw)
            w1_s = w1_ref[:, s * _CIN:(s + 1) * _CIN]
            wts = wts + jax.lax.dot_general(
                m_s, w1_s, (((1,), (1,)), ((), ())),
                preferred_element_type=jnp.float32)
        wts_ref[...] = wts

    @pl.when(i >= ht)
    def _():
        w = wts_ref[...].astype(jnp.bfloat16)
        th = thw // wdim
        for s in range(_S):
            for k in range(_B):
                xk = x_refs[s][k * _CIN:(k + 1) * _CIN, :]
                res = jnp.dot(w, xk, preferred_element_type=jnp.float32)
                o_refs[s][k * _B:(k + 1) * _B, :, :] = res.reshape(
                    _B, th, wdim)


def kernel(x0, x1, x2, x3, w1, b1):
    B, cin, H, W = x0.shape
    HW = H * W
    R = B * cin
    thw = min(_THW, HW)
    ht = HW // thw
    xs_c = [x.reshape(R, HW).astype(jnp.bfloat16) for x in (x0, x1, x2, x3)]

    body = functools.partial(_fused_kernel, ht=ht, thw=thw, hw=HW, wdim=W)

    def in_idx(i):
        return (0, jnp.where(i < ht, i, i - ht))

    def out_idx(i):
        return (0, jnp.where(i < ht, 0, i - ht), 0)

    th = thw // W
    out_shape = jax.ShapeDtypeStruct((B * B, H, W), jnp.float32)
    outs = pl.pallas_call(
        body,
        out_shape=[out_shape] * _S,
        grid=(2 * ht,),
        in_specs=[
            pl.BlockSpec((cin, _S * cin), lambda i: (0, 0)),
            pl.BlockSpec((1, cin), lambda i: (0, 0)),
        ] + [pl.BlockSpec((R, thw), in_idx)] * _S,
        out_specs=[pl.BlockSpec((B * B, th, W), out_idx)] * _S,
        scratch_shapes=[
            pltpu.VMEM((_S, R, _LANE), jnp.float32),
            pltpu.VMEM((_B, _CIN), jnp.float32),
        ],
        compiler_params=pltpu.CompilerParams(
            dimension_semantics=("arbitrary",),
            vmem_limit_bytes=48 * 1024 * 1024,
        ),
    )(w1, b1.reshape(1, cin), *xs_c)

    return [o.reshape(B, B, H, W) for o in outs]
